# baseline (speedup 1.0000x reference)
import functools
import jax
import jax.numpy as jnp
from jax import lax
from jax.experimental import pallas as pl
from jax.experimental.pallas import tpu as pltpu, tpu_sc as plsc

_NC, _NS = 2, 16
_NW = _NC * _NS
_L = 16
_HALF = 32 * 1024


def _sc_body(x_hbm, pe_hbm, out_hbm, pe_buf, x_buf, *, n_batch):
    wid = lax.axis_index("s") * _NC + lax.axis_index("c")
    pe0 = wid * (2 * _HALF)
    for p in range(2):
        pltpu.sync_copy(pe_hbm.at[pl.ds(pe0 + p * _HALF, _HALF)], pe_buf)
        for b in range(n_batch):
            r0 = b * (_NW * 2 * _HALF) + pe0 + p * _HALF
            pltpu.sync_copy(x_hbm.at[pl.ds(r0, _HALF)], x_buf)

            @plsc.parallel_loop(0, _HALF, step=_L, unroll=8)
            def _(k):
                plsc.addupdate(x_buf.at[pl.ds(k, _L)], pe_buf[pl.ds(k, _L)])

            pltpu.sync_copy(x_buf, out_hbm.at[pl.ds(r0, _HALF)])


def kernel(x, pe_table):
    B, S, D = x.shape
    mesh = plsc.VectorSubcoreMesh(core_axis_name="c", subcore_axis_name="s",
                                  num_cores=_NC, num_subcores=_NS)
    out = pl.kernel(
        functools.partial(_sc_body, n_batch=B),
        out_type=jax.ShapeDtypeStruct((B * S * D,), jnp.float32),
        mesh=mesh,
        scratch_types=[
            pltpu.VMEM((_HALF,), jnp.float32),
            pltpu.VMEM((_HALF,), jnp.float32),
        ],
    )(x.reshape(-1), pe_table.reshape(-1))
    return out.reshape(B, S, D)


# speedup vs baseline: 1.1280x; 1.1280x over previous
import functools
import jax
import jax.numpy as jnp
from jax import lax
from jax.experimental import pallas as pl
from jax.experimental.pallas import tpu as pltpu, tpu_sc as plsc

_NC, _NS = 2, 16
_NW = _NC * _NS
_L = 16
_HALF = 32 * 1024


def _sc_body(x_hbm, pe_hbm, out_hbm, pe_buf, xb0, xb1, l0, l1, s0, s1,
             *, n_batch):
    wid = lax.axis_index("s") * _NC + lax.axis_index("c")
    pe0 = wid * (2 * _HALF)
    xb = (xb0, xb1)
    lsem = (l0, l1)
    ssem = (s0, s1)
    pending_store = [None, None]

    def xoff(p, b):
        return b * (_NW * 2 * _HALF) + pe0 + p * _HALF

    def start_load(p, b):
        j = b % 2
        if pending_store[j] is not None:
            pending_store[j].wait()
            pending_store[j] = None
        return pltpu.async_copy(x_hbm.at[pl.ds(xoff(p, b), _HALF)], xb[j],
                                lsem[j])

    for p in range(2):
        pltpu.sync_copy(pe_hbm.at[pl.ds(pe0 + p * _HALF, _HALF)], pe_buf)
        load = start_load(p, 0)
        for b in range(n_batch):
            j = b % 2
            load.wait()
            if b + 1 < n_batch:
                load = start_load(p, b + 1)

            @plsc.parallel_loop(0, _HALF, step=_L, unroll=8)
            def _(k):
                plsc.addupdate(xb[j].at[pl.ds(k, _L)], pe_buf[pl.ds(k, _L)])

            pending_store[j] = pltpu.async_copy(
                xb[j], out_hbm.at[pl.ds(xoff(p, b), _HALF)], ssem[j])
    for j in range(2):
        if pending_store[j] is not None:
            pending_store[j].wait()


def kernel(x, pe_table):
    B, S, D = x.shape
    mesh = plsc.VectorSubcoreMesh(core_axis_name="c", subcore_axis_name="s",
                                  num_cores=_NC, num_subcores=_NS)
    out = pl.kernel(
        functools.partial(_sc_body, n_batch=B),
        out_type=jax.ShapeDtypeStruct((B * S * D,), jnp.float32),
        mesh=mesh,
        scratch_types=[
            pltpu.VMEM((_HALF,), jnp.float32),
            pltpu.VMEM((_HALF,), jnp.float32),
            pltpu.VMEM((_HALF,), jnp.float32),
            pltpu.SemaphoreType.DMA,
            pltpu.SemaphoreType.DMA,
            pltpu.SemaphoreType.DMA,
            pltpu.SemaphoreType.DMA,
        ],
    )(x.reshape(-1), pe_table.reshape(-1))
    return out.reshape(B, S, D)
